# Initial kernel scaffold; baseline (speedup 1.0000x reference)
import jax
import jax.numpy as jnp
from jax.experimental import pallas as pl


def kernel(s, v, p, edge_index_local, edge_index_global, edge_d, edge_a, edge_rnorm, edge_e, params, out_norm_params):
    raise NotImplementedError("write your pallas kernel here")


# speedup vs baseline: 14.3034x; 14.3034x over previous
import functools

import jax
import jax.numpy as jnp
from jax import lax
from jax.experimental import pallas as pl
from jax.experimental.pallas import tpu as pltpu
from jax.experimental.pallas import tpu_sc as plsc

SDIM = 64
VDIM = 16
EDGE_DIM = 16
CUTOFF = 5.0
MSGW = 128
CHUNK = 32
CH = 128
BN = 2000
BE = 4000
HI = jax.lax.Precision.HIGHEST


def _silu(x):
    return x / (1.0 + jnp.exp(-x))


def _envelope(d):
    return jnp.where(d < CUTOFF, 0.5 * (jnp.cos(jnp.pi * d / CUTOFF) + 1.0), 0.0)


def _layernorm(x, scale, bias):
    mu = jnp.mean(x, axis=-1, keepdims=True)
    var = jnp.mean((x - mu) ** 2, axis=-1, keepdims=True)
    return (x - mu) / jnp.sqrt(var + 1e-6) * scale + bias


def _vnorm48(v48, scale16):
    sq = v48[:, :16] ** 2 + v48[:, 16:32] ** 2 + v48[:, 32:48] ** 2
    n = jnp.sqrt(jnp.mean(sq, axis=-1, keepdims=True) + 1e-6)
    sc3 = jnp.concatenate([scale16, scale16, scale16], axis=-1)
    return v48 / n * sc3


def _prep0_body(s_ref, lns_ref, lnb_ref, w1a_ref, b1_ref, w1b_ref,
                sn_ref, xa_ref, xb_ref):
    sn = _layernorm(s_ref[...], lns_ref[...], lnb_ref[...])
    sn_ref[...] = sn
    xa_ref[...] = jnp.dot(sn, w1a_ref[...], precision=HI) + b1_ref[...]
    xb_ref[...] = jnp.dot(sn, w1b_ref[...], precision=HI)


def _edge_body(first, prea_ref, preb_ref, e_ref, x1_ref, x2_ref, x3_ref,
               wpre_ref, wout_ref, bout_ref, msg_ref, enew_ref):
    if first:
        d = x1_ref[...]
        a = x2_ref[...]
        rn = x3_ref[...]
    else:
        ps = x2_ref[...][:, :3]
        pd = x3_ref[...][:, :3]
        r = pd - ps
        a = jnp.sum(pd * ps, axis=-1, keepdims=True)
        d = jnp.sqrt(jnp.clip(jnp.sum(r * r, axis=-1, keepdims=True), 1e-6, None))
        rn = r / (1.0 + d)
    ones = jnp.ones_like(d)
    feat2 = jnp.concatenate([e_ref[...], d, a, ones], axis=-1)
    pre = prea_ref[...] + preb_ref[...] + jnp.dot(feat2, wpre_ref[...], precision=HI)
    sil = _silu(pre)
    C = _envelope(d)
    gout = jnp.dot(sil, wout_ref[...], precision=HI) + bout_ref[...]
    gr = gout[:, :VDIM] * C
    enew_ref[...] = gout[:, 2 * VDIM:3 * VDIM]
    pc = gout[:, 3 * VDIM:3 * VDIM + 1] * C
    if first:
        mv = [gr * rn[:, c:c + 1] for c in range(3)]
    else:
        gv = gout[:, VDIM:2 * VDIM] * C
        vs = x1_ref[...]
        mv = [gr * rn[:, c:c + 1] + gv * vs[:, 16 * c:16 * c + 16] for c in range(3)]
    pcr = pc * rn
    pad = jnp.zeros_like(pre[:, :11])
    msg_ref[...] = jnp.concatenate(
        [sil * C, mv[0], mv[1], mv[2], pcr, C, ones, pad], axis=-1)


def _upd0_body(acc_ref, s_ref, v_ref, p_ref, sn0_ref,
               w2ms_ref, b2ms_ref, wu1_ref, bu1_ref, wu2_ref, bu2_ref,
               lns_ref, lnb_ref, vns_ref, w1a_ref, b1_ref, w1b_ref,
               s1_ref, v1_ref, p1_ref, ppad_ref, xa_ref, xb_ref, vn1_ref):
    acc = acc_ref[...]
    inv = 1.0 / jnp.clip(acc[:, 116:117], 1.0, None)
    hsum = acc[:, :SDIM] * inv
    csum = acc[:, 115:116] * inv
    s_agg = jnp.dot(hsum, w2ms_ref[...], precision=HI) + csum * b2ms_ref[...]
    vagg = acc[:, SDIM:SDIM + 48] * inv
    p1 = p_ref[...] + acc[:, 112:115] * inv
    vn = jnp.sqrt(vagg[:, :16] ** 2 + vagg[:, 16:32] ** 2 + vagg[:, 32:48] ** 2 + 1e-6)
    uin = jnp.concatenate([sn0_ref[...], s_agg, vn], axis=-1)
    uh = _silu(jnp.dot(uin, wu1_ref[...], precision=HI) + bu1_ref[...])
    u = jnp.dot(uh, wu2_ref[...], precision=HI) + bu2_ref[...]
    s1 = s_ref[...] + u[:, :SDIM]
    gate = u[:, SDIM:SDIM + VDIM]
    g3 = jnp.concatenate([gate, gate, gate], axis=-1)
    v1 = v_ref[...] + g3 * vagg
    s1_ref[...] = s1
    v1_ref[...] = v1
    p1_ref[...] = p1
    ppad_ref[...] = jnp.concatenate([p1, jnp.zeros_like(acc[:, :5])], axis=-1)
    sn1 = _layernorm(s1, lns_ref[...], lnb_ref[...])
    xa_ref[...] = jnp.dot(sn1, w1a_ref[...], precision=HI) + b1_ref[...]
    xb_ref[...] = jnp.dot(sn1, w1b_ref[...], precision=HI)
    vn1_ref[...] = _vnorm48(v1, vns_ref[...])


def _upd1_body(acc_ref, s_ref, v_ref, p_ref, w2ms_ref, b2ms_ref,
               lns_ref, lnb_ref, vns_ref,
               so_ref, vo_ref, po_ref):
    acc = acc_ref[...]
    inv = 1.0 / jnp.clip(acc[:, 116:117], 1.0, None)
    hsum = acc[:, :SDIM] * inv
    csum = acc[:, 115:116] * inv
    s_agg = jnp.dot(hsum, w2ms_ref[...], precision=HI) + csum * b2ms_ref[...]
    vagg = acc[:, SDIM:SDIM + 48] * inv
    s2 = s_ref[...] + s_agg
    v2 = v_ref[...] + vagg
    po_ref[...] = p_ref[...] + acc[:, 112:115] * inv
    so_ref[...] = _layernorm(s2, lns_ref[...], lnb_ref[...])
    vo_ref[...] = _vnorm48(v2, vns_ref[...])


def _row_spec(bn, w):
    return pl.BlockSpec((bn, w), lambda i: (i, 0))


def _full_spec(shape):
    return pl.BlockSpec(shape, lambda i: tuple(0 for _ in shape))


def _sc_mesh():
    return plsc.VectorSubcoreMesh(core_axis_name="c", subcore_axis_name="s")


def _gather_kernel(nt, E):
    NB = E // CH
    rem = NB % 32
    full = NB // 32

    def body(widths, sels, *refs):
        tabs = refs[:nt]
        src_hbm, dst_hbm = refs[nt], refs[nt + 1]
        outs = refs[nt + 2:nt + 2 + nt]
        idx_s, idx_d = refs[nt + 2 + nt], refs[nt + 3 + nt]
        bufs = refs[nt + 4 + nt:nt + 4 + 2 * nt]
        sem = refs[nt + 4 + 2 * nt]
        wid = lax.axis_index("s") * 2 + lax.axis_index("c")
        nblk = jnp.where(wid < rem, full + 1, full)

        def step(i, _):
            base = (i * 32 + wid) * CH
            pltpu.sync_copy(src_hbm.at[pl.ds(base, CH)], idx_s)
            pltpu.sync_copy(dst_hbm.at[pl.ds(base, CH)], idx_d)
            for t in range(nt):
                idx = idx_s if sels[t] == 0 else idx_d
                pltpu.async_copy(tabs[t].at[idx], bufs[t], sem).wait()
                pltpu.sync_copy(bufs[t], outs[t].at[pl.ds(base, CH)])
            return 0

        lax.fori_loop(0, nblk, step, 0)

    def make(widths, sels):
        out_type = tuple(jax.ShapeDtypeStruct((E, w), jnp.float32) for w in widths)
        scratch = ([pltpu.VMEM((CH,), jnp.int32)] * 2
                   + [pltpu.VMEM((CH, w), jnp.float32) for w in widths]
                   + [pltpu.SemaphoreType.DMA])
        return pl.kernel(functools.partial(body, widths, sels),
                         out_type=out_type, mesh=_sc_mesh(),
                         compiler_params=pltpu.CompilerParams(use_tc_tiling_on_sc=False),
                         scratch_types=scratch)
    return make


def _scatter_kernel(N, E):
    NB = E // CH
    rem = NB % 16
    full = NB // 16
    rows_t = N // 16

    def body(msg_hbm, dst_hbm, zeros_hbm, out_hbm, mbuf, dbuf, acc_sh):
        cid = lax.axis_index("c")
        sid = lax.axis_index("s")
        nblk = jnp.where(sid < rem, full + 1, full)
        rbase = sid * rows_t

        for half in range(2):
            col = (cid * 2 + half) * CHUNK
            pltpu.sync_copy(zeros_hbm.at[pl.ds(rbase, rows_t)],
                            acc_sh.at[pl.ds(rbase, rows_t)])
            plsc.subcore_barrier()

            def step(i, _):
                base = (i * 16 + sid) * CH
                pltpu.sync_copy(dst_hbm.at[pl.ds(base, CH)], dbuf)
                pltpu.sync_copy(msg_hbm.at[pl.ds(base, CH), pl.ds(col, CHUNK)], mbuf)
                pltpu.sync_copy(mbuf, acc_sh.at[dbuf], add=True)
                return 0

            lax.fori_loop(0, nblk, step, 0)
            plsc.subcore_barrier()
            pltpu.sync_copy(acc_sh.at[pl.ds(rbase, rows_t)],
                            out_hbm.at[pl.ds(rbase, rows_t), pl.ds(col, CHUNK)])
            plsc.subcore_barrier()

    return pl.kernel(
        body,
        out_type=jax.ShapeDtypeStruct((N, MSGW), jnp.float32),
        mesh=_sc_mesh(),
        compiler_params=pltpu.CompilerParams(use_tc_tiling_on_sc=False),
        scratch_types=[pltpu.VMEM((CH, CHUNK), jnp.float32),
                       pltpu.VMEM((CH,), jnp.int32),
                       pltpu.VMEM_SHARED((N, CHUNK), jnp.float32)])


def kernel(s, v, p, edge_index_local, edge_index_global, edge_d, edge_a,
           edge_rnorm, edge_e, params, out_norm_params):
    N = s.shape[0]
    E = edge_e.shape[0]
    src = edge_index_global[0].astype(jnp.int32)
    dst = edge_index_global[1].astype(jnp.int32)
    v48 = v.reshape(N, 3 * VDIM)

    def packed(pr):
        W1, W2, b2 = pr['W1'], pr['W2'], pr['b2']
        wpre = jnp.concatenate(
            [W1[2 * SDIM:2 * SDIM + EDGE_DIM],
             W1[2 * SDIM + EDGE_DIM:2 * SDIM + EDGE_DIM + 1],
             W1[2 * SDIM + EDGE_DIM + 1:2 * SDIM + EDGE_DIM + 2],
             jnp.zeros((1, SDIM), jnp.float32)], axis=0)
        wout = W2[:, SDIM:]
        bout = b2[SDIM:].reshape(1, 49)
        return dict(
            w1a=W1[:SDIM], w1b=W1[SDIM:2 * SDIM],
            b1=pr['b1'].reshape(1, SDIM), wpre=wpre, wout=wout, bout=bout,
            w2ms=W2[:, :SDIM], b2ms=b2[:SDIM].reshape(1, SDIM),
            lns=pr['ln_scale'].reshape(1, SDIM), lnb=pr['ln_bias'].reshape(1, SDIM),
            vns=pr['vn_scale'].reshape(1, VDIM))

    pk0, pk1 = packed(params[0]), packed(params[1])
    nb_n = N // BN
    nb_e = E // BE

    sn0, xa0, xb0 = pl.pallas_call(
        _prep0_body,
        grid=(nb_n,),
        in_specs=[_row_spec(BN, SDIM), _full_spec((1, SDIM)), _full_spec((1, SDIM)),
                  _full_spec((SDIM, SDIM)), _full_spec((1, SDIM)), _full_spec((SDIM, SDIM))],
        out_specs=[_row_spec(BN, SDIM)] * 3,
        out_shape=[jax.ShapeDtypeStruct((N, SDIM), jnp.float32)] * 3,
    )(s, pk0['lns'], pk0['lnb'], pk0['w1a'], pk0['b1'], pk0['w1b'])

    g0 = _gather_kernel(2, E)((SDIM, SDIM), (0, 1))
    prea0, preb0 = g0(xa0, xb0, src, dst)

    d0 = edge_d.reshape(E, 1)
    a0 = edge_a.reshape(E, 1)
    msg0, e1 = pl.pallas_call(
        functools.partial(_edge_body, True),
        grid=(nb_e,),
        in_specs=[_row_spec(BE, SDIM), _row_spec(BE, SDIM), _row_spec(BE, EDGE_DIM),
                  _row_spec(BE, 1), _row_spec(BE, 1), _row_spec(BE, 3),
                  _full_spec((19, SDIM)), _full_spec((SDIM, 49)), _full_spec((1, 49))],
        out_specs=[_row_spec(BE, MSGW), _row_spec(BE, EDGE_DIM)],
        out_shape=[jax.ShapeDtypeStruct((E, MSGW), jnp.float32),
                   jax.ShapeDtypeStruct((E, EDGE_DIM), jnp.float32)],
    )(prea0, preb0, edge_e, d0, a0, edge_rnorm,
      pk0['wpre'], pk0['wout'], pk0['bout'])

    zeros_acc = jnp.zeros((N, CHUNK), jnp.float32)
    scat = _scatter_kernel(N, E)
    acc0 = scat(msg0, dst, zeros_acc)

    pr0 = params[0]
    s1, v1, p1, ppad, xa1, xb1, vn1 = pl.pallas_call(
        _upd0_body,
        grid=(nb_n,),
        in_specs=[_row_spec(BN, MSGW), _row_spec(BN, SDIM), _row_spec(BN, 48),
                  _row_spec(BN, 3), _row_spec(BN, SDIM),
                  _full_spec((SDIM, SDIM)), _full_spec((1, SDIM)),
                  _full_spec((144, SDIM)), _full_spec((1, SDIM)),
                  _full_spec((SDIM, 80)), _full_spec((1, 80)),
                  _full_spec((1, SDIM)), _full_spec((1, SDIM)), _full_spec((1, VDIM)),
                  _full_spec((SDIM, SDIM)), _full_spec((1, SDIM)), _full_spec((SDIM, SDIM))],
        out_specs=[_row_spec(BN, SDIM), _row_spec(BN, 48), _row_spec(BN, 3),
                   _row_spec(BN, 8), _row_spec(BN, SDIM), _row_spec(BN, SDIM),
                   _row_spec(BN, 48)],
        out_shape=[jax.ShapeDtypeStruct((N, SDIM), jnp.float32),
                   jax.ShapeDtypeStruct((N, 48), jnp.float32),
                   jax.ShapeDtypeStruct((N, 3), jnp.float32),
                   jax.ShapeDtypeStruct((N, 8), jnp.float32),
                   jax.ShapeDtypeStruct((N, SDIM), jnp.float32),
                   jax.ShapeDtypeStruct((N, SDIM), jnp.float32),
                   jax.ShapeDtypeStruct((N, 48), jnp.float32)],
    )(acc0, s, v48, p, sn0,
      pk0['w2ms'], pk0['b2ms'],
      pr0['Wu1'], pr0['bu1'].reshape(1, SDIM),
      pr0['Wu2'], pr0['bu2'].reshape(1, 80),
      pk1['lns'], pk1['lnb'], pk1['vns'], pk1['w1a'], pk1['b1'], pk1['w1b'])

    g1 = _gather_kernel(5, E)((SDIM, SDIM, 48, 8, 8), (0, 1, 0, 0, 1))
    prea1, preb1, vsrc1, ps1, pd1 = g1(xa1, xb1, vn1, ppad, ppad, src, dst)

    msg1, e2 = pl.pallas_call(
        functools.partial(_edge_body, False),
        grid=(nb_e,),
        in_specs=[_row_spec(BE, SDIM), _row_spec(BE, SDIM), _row_spec(BE, EDGE_DIM),
                  _row_spec(BE, 48), _row_spec(BE, 8), _row_spec(BE, 8),
                  _full_spec((19, SDIM)), _full_spec((SDIM, 49)), _full_spec((1, 49))],
        out_specs=[_row_spec(BE, MSGW), _row_spec(BE, EDGE_DIM)],
        out_shape=[jax.ShapeDtypeStruct((E, MSGW), jnp.float32),
                   jax.ShapeDtypeStruct((E, EDGE_DIM), jnp.float32)],
    )(prea1, preb1, e1, vsrc1, ps1, pd1,
      pk1['wpre'], pk1['wout'], pk1['bout'])

    acc1 = scat(msg1, dst, zeros_acc)

    onp = out_norm_params
    so, vo, po = pl.pallas_call(
        _upd1_body,
        grid=(nb_n,),
        in_specs=[_row_spec(BN, MSGW), _row_spec(BN, SDIM), _row_spec(BN, 48),
                  _row_spec(BN, 3),
                  _full_spec((SDIM, SDIM)), _full_spec((1, SDIM)),
                  _full_spec((1, SDIM)), _full_spec((1, SDIM)), _full_spec((1, VDIM))],
        out_specs=[_row_spec(BN, SDIM), _row_spec(BN, 48), _row_spec(BN, 3)],
        out_shape=[jax.ShapeDtypeStruct((N, SDIM), jnp.float32),
                   jax.ShapeDtypeStruct((N, 48), jnp.float32),
                   jax.ShapeDtypeStruct((N, 3), jnp.float32)],
    )(acc1, s1, v1, p1, pk1['w2ms'], pk1['b2ms'],
      onp['ln_scale'].reshape(1, SDIM), onp['ln_bias'].reshape(1, SDIM),
      onp['vn_scale'].reshape(1, VDIM))

    return so, vo.reshape(N, 3, VDIM), e2, po
